# baseline (speedup 1.0000x reference)
import functools

import jax
import jax.numpy as jnp
from jax import lax
from jax.experimental import pallas as pl
from jax.experimental.pallas import tpu as pltpu
from jax.experimental.pallas import tpu_sc as plsc

N = 10000
E = 320000
D = 128
K = 1024
NPAD = 10112
NC = NPAD // 128
BLK = 632
NBLK = NPAD // BLK

NCORES = 2
NSUB = 16
NW = NCORES * NSUB
CHUNK = 112
NCHUNK = -(-(E // NW) // CHUNK)
EDGES_PER_W = NCHUNK * CHUNK
NBUF = 2
ROWS_PER_SUB = NPAD // NSUB


def _sc_scatter_body(x_hbm, src_hbm, dst_hbm, zeros_hbm, out_hbm,
                     idx_s, idx_d, rows, acc, sem, isem):
    c = lax.axis_index("c")
    s = lax.axis_index("s")
    wid = c * NSUB + s

    ibase = pl.multiple_of(wid * EDGES_PER_W, 8)
    src_cp = pltpu.async_copy(src_hbm.at[pl.ds(ibase, EDGES_PER_W)], idx_s,
                              isem)
    dst_cp = pltpu.async_copy(dst_hbm.at[pl.ds(ibase, EDGES_PER_W)], idx_d,
                              isem)

    zoff = pl.multiple_of(s * ROWS_PER_SUB, 8)
    pltpu.sync_copy(zeros_hbm, acc.at[pl.ds(zoff, ROWS_PER_SUB)])
    src_cp.wait()
    dst_cp.wait()
    plsc.subcore_barrier()

    pltpu.async_copy(x_hbm.at[idx_s.at[pl.ds(0, CHUNK)]], rows.at[0], sem)

    def chunk_body(k, carry):
        buf = lax.rem(k, NBUF)
        nbuf = lax.rem(k + 1, NBUF)

        @pl.when(k + 1 < NCHUNK)
        def _():
            pltpu.async_copy(x_hbm.at[idx_s.at[pl.ds((k + 1) * CHUNK, CHUNK)]],
                             rows.at[nbuf], sem)

        pltpu.make_async_copy(x_hbm.at[idx_s.at[pl.ds(k * CHUNK, CHUNK)]],
                              rows.at[buf], sem).wait()
        pltpu.sync_copy(rows.at[buf],
                        acc.at[idx_d.at[pl.ds(k * CHUNK, CHUNK)]], add=True)
        return carry

    lax.fori_loop(0, NCHUNK, chunk_body, 0)
    plsc.subcore_barrier()

    ooff = pl.multiple_of(c * NPAD + s * ROWS_PER_SUB, 8)
    pltpu.sync_copy(acc.at[pl.ds(zoff, ROWS_PER_SUB)],
                    out_hbm.at[pl.ds(ooff, ROWS_PER_SUB)])


@functools.cache
def _sc_scatter_fn():
    return pl.kernel(
        _sc_scatter_body,
        out_type=jax.ShapeDtypeStruct((NCORES * NPAD, D), jnp.float32),
        mesh=plsc.VectorSubcoreMesh(core_axis_name="c", subcore_axis_name="s",
                                    num_cores=NCORES, num_subcores=NSUB),
        scratch_types=[
            pltpu.VMEM((EDGES_PER_W,), jnp.int32),
            pltpu.VMEM((EDGES_PER_W,), jnp.int32),
            pltpu.VMEM((NBUF, CHUNK, D), jnp.float32),
            pltpu.VMEM_SHARED((NPAD, D), jnp.float32),
            pltpu.SemaphoreType.DMA,
            pltpu.SemaphoreType.DMA,
        ],
    )


def _sc_scatter(*args):
    return _sc_scatter_fn()(*args)


def _gin_layer_kernel(x_ref, a0_ref, a1_ref, wa_ref, ba_ref, wb_ref, bb_ref,
                      o_ref):
    h = x_ref[...] + a0_ref[...] + a1_ref[...]
    t = jnp.dot(h, wa_ref[...], preferred_element_type=jnp.float32)
    t = jnp.maximum(t + ba_ref[...], 0.0)
    o = jnp.dot(t, wb_ref[...], preferred_element_type=jnp.float32)
    o_ref[...] = jnp.maximum(o + bb_ref[...], 0.0)


_gin_layer = pl.pallas_call(
    _gin_layer_kernel,
    grid=(NBLK,),
    in_specs=[
        pl.BlockSpec((BLK, D), lambda i: (i, 0)),
        pl.BlockSpec((BLK, D), lambda i: (i, 0)),
        pl.BlockSpec((BLK, D), lambda i: (i + NBLK, 0)),
        pl.BlockSpec((D, D), lambda i: (0, 0)),
        pl.BlockSpec((1, D), lambda i: (0, 0)),
        pl.BlockSpec((D, D), lambda i: (0, 0)),
        pl.BlockSpec((1, D), lambda i: (0, 0)),
    ],
    out_specs=pl.BlockSpec((BLK, D), lambda i: (i, 0)),
    out_shape=jax.ShapeDtypeStruct((NPAD, D), jnp.float32),
)


def _ln(v, g, b):
    m = jnp.mean(v, axis=-1, keepdims=True)
    var = jnp.mean((v - m) ** 2, axis=-1, keepdims=True)
    return (v - m) / jnp.sqrt(var + 1e-5) * g + b


def _head_kernel(x1_ref, x2_ref, wl_ref, bl_ref, f1w_ref, f1b_ref, f2w_ref,
                 f2b_ref, f3w_ref, f3b_ref, g1_ref, be1_ref, g2_ref, be2_ref,
                 o_ref, sim_ref, rank_ref, cnt_ref, sums_ref):
    o1 = jnp.dot(x1_ref[...], wl_ref[...], preferred_element_type=jnp.float32)
    o2 = jnp.dot(x2_ref[...], wl_ref[...], preferred_element_type=jnp.float32)
    dd = (o1 + bl_ref[...]) - (o2 + bl_ref[...]) + 1e-6
    sim = jnp.sqrt(jnp.sum(dd * dd, axis=-1))
    sim2 = sim.reshape(NC, 128)
    idx2 = (lax.broadcasted_iota(jnp.int32, (NC, 128), 0) * 128
            + lax.broadcasted_iota(jnp.int32, (NC, 128), 1))
    sim2 = jnp.where(idx2 < N, sim2, -1.0)
    sim_ref[...] = sim2
    rank_ref[...] = jnp.zeros((NC, 128), jnp.float32)

    s_i = sim2.reshape(NC, 128, 1)

    def rank_body(c, carry):
        s_j = sim_ref[pl.ds(c, 1), :].reshape(1, 1, 128)
        gt = (s_j > s_i).astype(jnp.float32)
        rank_ref[...] += jnp.sum(gt, axis=2)
        return carry

    lax.fori_loop(0, NC, rank_body, 0)

    r_i = rank_ref[...].reshape(NC, 128, 1)

    def val_body(rc, carry):
        rr = (rc * 128
              + lax.broadcasted_iota(jnp.int32, (1, 1, 128), 2)
              ).astype(jnp.float32)
        hit = (r_i == rr).astype(jnp.float32)
        ct = jnp.sum(jnp.sum(hit, axis=0), axis=0)
        sm = jnp.sum(jnp.sum(hit * s_i, axis=0), axis=0)
        cnt_ref[pl.ds(rc, 1), :] = ct.reshape(1, 128)
        sums_ref[pl.ds(rc, 1), :] = sm.reshape(1, 128)
        return carry

    lax.fori_loop(0, K // 128, val_body, 0)

    cnt_row = cnt_ref[...].reshape(1, K)
    val_row = sums_ref[...].reshape(1, K) / jnp.maximum(cnt_row, 1.0)
    t_col = lax.broadcasted_iota(jnp.int32, (K, K), 0).astype(jnp.float32)
    r_row = lax.broadcasted_iota(jnp.int32, (K, K), 1).astype(jnp.float32)
    m2 = ((r_row <= t_col) & (t_col < r_row + cnt_row)).astype(jnp.float32)
    b = lax.dot_general(m2, f1w_ref[...], (((0,), (0,)), ((), ())),
                        preferred_element_type=jnp.float32)
    h = jnp.dot(val_row, b, preferred_element_type=jnp.float32)
    h = jnp.maximum(_ln(h + f1b_ref[...], g1_ref[...], be1_ref[...]), 0.0)
    h = jnp.dot(h, f2w_ref[...], preferred_element_type=jnp.float32)
    h = jnp.maximum(_ln(h + f2b_ref[...], g2_ref[...], be2_ref[...]), 0.0)
    h = jnp.dot(h, f3w_ref[...], preferred_element_type=jnp.float32)
    h = h + f3b_ref[...]
    o_ref[...] = 1.0 / (1.0 + jnp.exp(-h))


_head = pl.pallas_call(
    _head_kernel,
    out_shape=jax.ShapeDtypeStruct((1, 1), jnp.float32),
    scratch_shapes=[
        pltpu.VMEM((NC, 128), jnp.float32),
        pltpu.VMEM((NC, 128), jnp.float32),
        pltpu.VMEM((K // 128, 128), jnp.float32),
        pltpu.VMEM((K // 128, 128), jnp.float32),
    ],
)


def _gnn(xp, src, dst, zeros_rows, p, sfx):
    for l in ("1", "2", "3"):
        agg = _sc_scatter(xp, src, dst, zeros_rows)
        xp = _gin_layer(xp, agg, agg, p["w" + l + "a"], p["b" + l + "a"],
                        p["w" + l + "b"], p["b" + l + "b"])
    return xp


def kernel(x1, x2, edge_index1, edge_index2, w1a, b1a, w1b, b1b, w2a, b2a,
           w2b, b2b, w3a, b3a, w3b, b3b, wl, bl, fc1_w, fc1_b, fc2_w, fc2_b,
           fc3_w, fc3_b, g1, be1, g2, be2):
    f32 = jnp.float32
    xp1 = jnp.pad(x1, ((0, NPAD - N), (0, 0)))
    xp2 = jnp.pad(x2, ((0, NPAD - N), (0, 0)))
    zeros_rows = jnp.zeros((ROWS_PER_SUB, D), f32)
    p = {"w1a": w1a, "b1a": b1a.reshape(1, D), "w1b": w1b,
         "b1b": b1b.reshape(1, D), "w2a": w2a, "b2a": b2a.reshape(1, D),
         "w2b": w2b, "b2b": b2b.reshape(1, D), "w3a": w3a,
         "b3a": b3a.reshape(1, D), "w3b": w3b, "b3b": b3b.reshape(1, D)}

    padw = EDGES_PER_W - E // NW

    def _pad_edges(v, fill):
        return jnp.pad(v.reshape(NW, E // NW), ((0, 0), (0, padw)),
                       constant_values=fill).reshape(-1)

    e1s = _pad_edges(edge_index1[0], N)
    e1d = _pad_edges(edge_index1[1], NPAD - 1)
    e2s = _pad_edges(edge_index2[0], N)
    e2d = _pad_edges(edge_index2[1], NPAD - 1)
    h1 = _gnn(xp1, e1s, e1d, zeros_rows, p, "g1")
    h2 = _gnn(xp2, e2s, e2d, zeros_rows, p, "g2")

    out = _head(h1, h2, wl, bl.reshape(1, 2), fc1_w, fc1_b.reshape(1, D),
                fc2_w, fc2_b.reshape(1, D), fc3_w, fc3_b.reshape(1, 1),
                g1.reshape(1, D), be1.reshape(1, D), g2.reshape(1, D),
                be2.reshape(1, D))
    return out.reshape(1)


# speedup vs baseline: 1.5483x; 1.5483x over previous
import functools

import jax
import jax.numpy as jnp
from jax import lax
from jax.experimental import pallas as pl
from jax.experimental.pallas import tpu as pltpu
from jax.experimental.pallas import tpu_sc as plsc

N = 10000
E = 320000
D = 128
K = 1024
NPAD = 10112
NC = NPAD // 128
BLK = 632
NBLK = NPAD // BLK

NCORES = 2
NSUB = 16
NW = NCORES * NSUB
CHUNK = 80
NCHUNK = -(-(E // NW) // CHUNK)
EDGES_PER_W = NCHUNK * CHUNK
NBUF = 2
ROWS_PER_SUB = NPAD // NSUB


def _sc_scatter_body(x_hbm, src_hbm, dst_hbm, zeros_hbm, out_hbm,
                     idx_s, idx_d, rows, acc, sem, isem):
    c = lax.axis_index("c")
    s = lax.axis_index("s")
    wid = c * NSUB + s

    ibase = pl.multiple_of(wid * EDGES_PER_W, 8)
    src_cp = pltpu.async_copy(src_hbm.at[pl.ds(ibase, EDGES_PER_W)], idx_s,
                              isem)
    dst_cp = pltpu.async_copy(dst_hbm.at[pl.ds(ibase, EDGES_PER_W)], idx_d,
                              isem)

    zoff = pl.multiple_of(s * ROWS_PER_SUB, 8)
    pltpu.sync_copy(zeros_hbm, acc.at[pl.ds(zoff, ROWS_PER_SUB)])
    src_cp.wait()
    dst_cp.wait()
    plsc.subcore_barrier()

    pltpu.async_copy(x_hbm.at[idx_s.at[pl.ds(0, CHUNK)]], rows.at[0], sem)

    def chunk_body(k, carry):
        buf = lax.rem(k, NBUF)
        nbuf = lax.rem(k + 1, NBUF)

        @pl.when(k + 1 < NCHUNK)
        def _():
            pltpu.async_copy(x_hbm.at[idx_s.at[pl.ds((k + 1) * CHUNK, CHUNK)]],
                             rows.at[nbuf], sem)

        pltpu.make_async_copy(x_hbm.at[idx_s.at[pl.ds(k * CHUNK, CHUNK)]],
                              rows.at[buf], sem).wait()
        pltpu.sync_copy(rows.at[buf],
                        acc.at[idx_d.at[pl.ds(k * CHUNK, CHUNK)]], add=True)
        return carry

    lax.fori_loop(0, NCHUNK, chunk_body, 0)
    plsc.subcore_barrier()

    ooff = pl.multiple_of(c * NPAD + s * ROWS_PER_SUB, 8)
    pltpu.sync_copy(acc.at[pl.ds(zoff, ROWS_PER_SUB)],
                    out_hbm.at[pl.ds(ooff, ROWS_PER_SUB)])


@functools.cache
def _sc_scatter_fn():
    return pl.kernel(
        _sc_scatter_body,
        out_type=jax.ShapeDtypeStruct((NCORES * NPAD, D), jnp.float32),
        mesh=plsc.VectorSubcoreMesh(core_axis_name="c", subcore_axis_name="s",
                                    num_cores=NCORES, num_subcores=NSUB),
        scratch_types=[
            pltpu.VMEM((EDGES_PER_W,), jnp.int32),
            pltpu.VMEM((EDGES_PER_W,), jnp.int32),
            pltpu.VMEM((NBUF, CHUNK, D), jnp.float32),
            pltpu.VMEM_SHARED((NPAD, D), jnp.float32),
            pltpu.SemaphoreType.DMA,
            pltpu.SemaphoreType.DMA,
        ],
    )


def _sc_scatter(*args):
    return _sc_scatter_fn()(*args)


def _gin_layer_kernel(x_ref, a0_ref, a1_ref, wa_ref, ba_ref, wb_ref, bb_ref,
                      o_ref):
    h = x_ref[...] + a0_ref[...] + a1_ref[...]
    t = jnp.dot(h, wa_ref[...], preferred_element_type=jnp.float32)
    t = jnp.maximum(t + ba_ref[...], 0.0)
    o = jnp.dot(t, wb_ref[...], preferred_element_type=jnp.float32)
    o_ref[...] = jnp.maximum(o + bb_ref[...], 0.0)


_gin_layer = pl.pallas_call(
    _gin_layer_kernel,
    grid=(NBLK,),
    in_specs=[
        pl.BlockSpec((BLK, D), lambda i: (i, 0)),
        pl.BlockSpec((BLK, D), lambda i: (i, 0)),
        pl.BlockSpec((BLK, D), lambda i: (i + NBLK, 0)),
        pl.BlockSpec((D, D), lambda i: (0, 0)),
        pl.BlockSpec((1, D), lambda i: (0, 0)),
        pl.BlockSpec((D, D), lambda i: (0, 0)),
        pl.BlockSpec((1, D), lambda i: (0, 0)),
    ],
    out_specs=pl.BlockSpec((BLK, D), lambda i: (i, 0)),
    out_shape=jax.ShapeDtypeStruct((NPAD, D), jnp.float32),
)


def _ln(v, g, b):
    m = jnp.mean(v, axis=-1, keepdims=True)
    var = jnp.mean((v - m) ** 2, axis=-1, keepdims=True)
    return (v - m) / jnp.sqrt(var + 1e-5) * g + b


def _head_kernel(x1_ref, x2_ref, wl_ref, bl_ref, f1w_ref, f1b_ref, f2w_ref,
                 f2b_ref, f3w_ref, f3b_ref, g1_ref, be1_ref, g2_ref, be2_ref,
                 o_ref, sim_ref, rank_ref, cnt_ref, sums_ref):
    o1 = jnp.dot(x1_ref[...], wl_ref[...], preferred_element_type=jnp.float32)
    o2 = jnp.dot(x2_ref[...], wl_ref[...], preferred_element_type=jnp.float32)
    dd = (o1 + bl_ref[...]) - (o2 + bl_ref[...]) + 1e-6
    sim = jnp.sqrt(jnp.sum(dd * dd, axis=-1))
    sim2 = sim.reshape(NC, 128)
    idx2 = (lax.broadcasted_iota(jnp.int32, (NC, 128), 0) * 128
            + lax.broadcasted_iota(jnp.int32, (NC, 128), 1))
    sim2 = jnp.where(idx2 < N, sim2, -1.0)
    sim_ref[...] = sim2
    rank_ref[...] = jnp.zeros((NC, 128), jnp.float32)

    s_i = sim2.reshape(NC, 128, 1)

    def rank_body(c, carry):
        s_j = sim_ref[pl.ds(c, 1), :].reshape(1, 1, 128)
        gt = (s_j > s_i).astype(jnp.float32)
        rank_ref[...] += jnp.sum(gt, axis=2)
        return carry

    lax.fori_loop(0, NC, rank_body, 0)

    r_i = rank_ref[...].reshape(NC, 128, 1)

    def val_body(rc, carry):
        rr = (rc * 128
              + lax.broadcasted_iota(jnp.int32, (1, 1, 128), 2)
              ).astype(jnp.float32)
        hit = (r_i == rr).astype(jnp.float32)
        ct = jnp.sum(jnp.sum(hit, axis=0), axis=0)
        sm = jnp.sum(jnp.sum(hit * s_i, axis=0), axis=0)
        cnt_ref[pl.ds(rc, 1), :] = ct.reshape(1, 128)
        sums_ref[pl.ds(rc, 1), :] = sm.reshape(1, 128)
        return carry

    lax.fori_loop(0, K // 128, val_body, 0)

    cnt_row = cnt_ref[...].reshape(1, K)
    val_row = sums_ref[...].reshape(1, K) / jnp.maximum(cnt_row, 1.0)
    t_col = lax.broadcasted_iota(jnp.int32, (K, K), 0).astype(jnp.float32)
    r_row = lax.broadcasted_iota(jnp.int32, (K, K), 1).astype(jnp.float32)
    m2 = ((r_row <= t_col) & (t_col < r_row + cnt_row)).astype(jnp.float32)
    b = lax.dot_general(m2, f1w_ref[...], (((0,), (0,)), ((), ())),
                        preferred_element_type=jnp.float32)
    h = jnp.dot(val_row, b, preferred_element_type=jnp.float32)
    h = jnp.maximum(_ln(h + f1b_ref[...], g1_ref[...], be1_ref[...]), 0.0)
    h = jnp.dot(h, f2w_ref[...], preferred_element_type=jnp.float32)
    h = jnp.maximum(_ln(h + f2b_ref[...], g2_ref[...], be2_ref[...]), 0.0)
    h = jnp.dot(h, f3w_ref[...], preferred_element_type=jnp.float32)
    h = h + f3b_ref[...]
    o_ref[...] = 1.0 / (1.0 + jnp.exp(-h))


_head = pl.pallas_call(
    _head_kernel,
    out_shape=jax.ShapeDtypeStruct((1, 1), jnp.float32),
    scratch_shapes=[
        pltpu.VMEM((NC, 128), jnp.float32),
        pltpu.VMEM((NC, 128), jnp.float32),
        pltpu.VMEM((K // 128, 128), jnp.float32),
        pltpu.VMEM((K // 128, 128), jnp.float32),
    ],
)


def _gnn(xp, src, dst, zeros_rows, p, sfx):
    for l in ("1", "2", "3"):
        agg = _sc_scatter(xp, src, dst, zeros_rows)
        xp = _gin_layer(xp, agg, agg, p["w" + l + "a"], p["b" + l + "a"],
                        p["w" + l + "b"], p["b" + l + "b"])
    return xp


def kernel(x1, x2, edge_index1, edge_index2, w1a, b1a, w1b, b1b, w2a, b2a,
           w2b, b2b, w3a, b3a, w3b, b3b, wl, bl, fc1_w, fc1_b, fc2_w, fc2_b,
           fc3_w, fc3_b, g1, be1, g2, be2):
    f32 = jnp.float32
    xp1 = jnp.pad(x1, ((0, NPAD - N), (0, 0)))
    xp2 = jnp.pad(x2, ((0, NPAD - N), (0, 0)))
    zeros_rows = jnp.zeros((ROWS_PER_SUB, D), f32)
    p = {"w1a": w1a, "b1a": b1a.reshape(1, D), "w1b": w1b,
         "b1b": b1b.reshape(1, D), "w2a": w2a, "b2a": b2a.reshape(1, D),
         "w2b": w2b, "b2b": b2b.reshape(1, D), "w3a": w3a,
         "b3a": b3a.reshape(1, D), "w3b": w3b, "b3b": b3b.reshape(1, D)}

    e1s = edge_index1[0]
    e1d = edge_index1[1]
    e2s = edge_index2[0]
    e2d = edge_index2[1]
    h1 = _gnn(xp1, e1s, e1d, zeros_rows, p, "g1")
    h2 = _gnn(xp2, e2s, e2d, zeros_rows, p, "g2")

    out = _head(h1, h2, wl, bl.reshape(1, 2), fc1_w, fc1_b.reshape(1, D),
                fc2_w, fc2_b.reshape(1, D), fc3_w, fc3_b.reshape(1, 1),
                g1.reshape(1, D), be1.reshape(1, D), g2.reshape(1, D),
                be2.reshape(1, D))
    return out.reshape(1)


# speedup vs baseline: 1.5695x; 1.0137x over previous
import functools

import jax
import jax.numpy as jnp
from jax import lax
from jax.experimental import pallas as pl
from jax.experimental.pallas import tpu as pltpu
from jax.experimental.pallas import tpu_sc as plsc

N = 10000
E = 320000
D = 128
K = 1024
NPAD = 10112
NC = NPAD // 128
BLK = 632
NBLK = NPAD // BLK

NCORES = 2
NSUB = 16
NW = NCORES * NSUB
CHUNK = 80
NCHUNK = -(-(E // NW) // CHUNK)
EDGES_PER_W = NCHUNK * CHUNK
NBUF = 2
ROWS_PER_SUB = NPAD // NSUB


def _sc_scatter_body(x_hbm, edges_hbm, zeros_hbm, out_hbm,
                     idx_s, idx_d, rows, acc, sem, isem):
    c = lax.axis_index("c")
    s = lax.axis_index("s")
    wid = c * NSUB + s

    ibase = pl.multiple_of(wid * EDGES_PER_W, 8)
    src_cp = pltpu.async_copy(edges_hbm.at[pl.ds(ibase, EDGES_PER_W)], idx_s,
                              isem)
    dst_cp = pltpu.async_copy(edges_hbm.at[pl.ds(E + ibase, EDGES_PER_W)],
                              idx_d, isem)

    zoff = pl.multiple_of(s * ROWS_PER_SUB, 8)
    pltpu.sync_copy(zeros_hbm, acc.at[pl.ds(zoff, ROWS_PER_SUB)])
    src_cp.wait()
    dst_cp.wait()
    plsc.subcore_barrier()

    pltpu.async_copy(x_hbm.at[idx_s.at[pl.ds(0, CHUNK)]], rows.at[0], sem)

    def chunk_body(k, carry):
        buf = lax.rem(k, NBUF)
        nbuf = lax.rem(k + 1, NBUF)

        @pl.when(k + 1 < NCHUNK)
        def _():
            pltpu.async_copy(x_hbm.at[idx_s.at[pl.ds((k + 1) * CHUNK, CHUNK)]],
                             rows.at[nbuf], sem)

        pltpu.make_async_copy(x_hbm.at[idx_s.at[pl.ds(k * CHUNK, CHUNK)]],
                              rows.at[buf], sem).wait()
        pltpu.sync_copy(rows.at[buf],
                        acc.at[idx_d.at[pl.ds(k * CHUNK, CHUNK)]], add=True)
        return carry

    lax.fori_loop(0, NCHUNK, chunk_body, 0)
    plsc.subcore_barrier()

    ooff = pl.multiple_of(c * NPAD + s * ROWS_PER_SUB, 8)
    pltpu.sync_copy(acc.at[pl.ds(zoff, ROWS_PER_SUB)],
                    out_hbm.at[pl.ds(ooff, ROWS_PER_SUB)])


@functools.cache
def _sc_scatter_fn():
    return pl.kernel(
        _sc_scatter_body,
        out_type=jax.ShapeDtypeStruct((NCORES * NPAD, D), jnp.float32),
        mesh=plsc.VectorSubcoreMesh(core_axis_name="c", subcore_axis_name="s",
                                    num_cores=NCORES, num_subcores=NSUB),
        scratch_types=[
            pltpu.VMEM((EDGES_PER_W,), jnp.int32),
            pltpu.VMEM((EDGES_PER_W,), jnp.int32),
            pltpu.VMEM((NBUF, CHUNK, D), jnp.float32),
            pltpu.VMEM_SHARED((NPAD, D), jnp.float32),
            pltpu.SemaphoreType.DMA,
            pltpu.SemaphoreType.DMA,
        ],
    )


def _sc_scatter(*args):
    return _sc_scatter_fn()(*args)


def _gin_layer_kernel(x_ref, a0_ref, a1_ref, wa_ref, ba_ref, wb_ref, bb_ref,
                      o_ref):
    h = x_ref[...] + a0_ref[...] + a1_ref[...]
    t = jnp.dot(h, wa_ref[...], preferred_element_type=jnp.float32)
    t = jnp.maximum(t + ba_ref[...], 0.0)
    o = jnp.dot(t, wb_ref[...], preferred_element_type=jnp.float32)
    o_ref[...] = jnp.maximum(o + bb_ref[...], 0.0)


_gin_layer = pl.pallas_call(
    _gin_layer_kernel,
    grid=(NBLK,),
    in_specs=[
        pl.BlockSpec((BLK, D), lambda i: (i, 0)),
        pl.BlockSpec((BLK, D), lambda i: (i, 0)),
        pl.BlockSpec((BLK, D), lambda i: (i + NBLK, 0)),
        pl.BlockSpec((D, D), lambda i: (0, 0)),
        pl.BlockSpec((1, D), lambda i: (0, 0)),
        pl.BlockSpec((D, D), lambda i: (0, 0)),
        pl.BlockSpec((1, D), lambda i: (0, 0)),
    ],
    out_specs=pl.BlockSpec((BLK, D), lambda i: (i, 0)),
    out_shape=jax.ShapeDtypeStruct((NPAD, D), jnp.float32),
)


def _ln(v, g, b):
    m = jnp.mean(v, axis=-1, keepdims=True)
    var = jnp.mean((v - m) ** 2, axis=-1, keepdims=True)
    return (v - m) / jnp.sqrt(var + 1e-5) * g + b


def _head_kernel(x1_ref, x2_ref, wl_ref, bl_ref, f1w_ref, f1b_ref, f2w_ref,
                 f2b_ref, f3w_ref, f3b_ref, g1_ref, be1_ref, g2_ref, be2_ref,
                 o_ref, sim_ref, rank_ref, cnt_ref, sums_ref):
    o1 = jnp.dot(x1_ref[...], wl_ref[...], preferred_element_type=jnp.float32)
    o2 = jnp.dot(x2_ref[...], wl_ref[...], preferred_element_type=jnp.float32)
    dd = (o1 + bl_ref[...]) - (o2 + bl_ref[...]) + 1e-6
    sim = jnp.sqrt(jnp.sum(dd * dd, axis=-1))
    sim2 = sim.reshape(NC, 128)
    idx2 = (lax.broadcasted_iota(jnp.int32, (NC, 128), 0) * 128
            + lax.broadcasted_iota(jnp.int32, (NC, 128), 1))
    sim2 = jnp.where(idx2 < N, sim2, -1.0)
    sim_ref[...] = sim2
    rank_ref[...] = jnp.zeros((NC, 128), jnp.float32)

    s_i = sim2.reshape(NC, 128, 1)

    def rank_body(c, carry):
        s_j = sim_ref[pl.ds(c, 1), :].reshape(1, 1, 128)
        gt = (s_j > s_i).astype(jnp.float32)
        rank_ref[...] += jnp.sum(gt, axis=2)
        return carry

    lax.fori_loop(0, NC, rank_body, 0)

    r_i = rank_ref[...].reshape(NC, 128, 1)

    def val_body(rc, carry):
        rr = (rc * 128
              + lax.broadcasted_iota(jnp.int32, (1, 1, 128), 2)
              ).astype(jnp.float32)
        hit = (r_i == rr).astype(jnp.float32)
        ct = jnp.sum(jnp.sum(hit, axis=0), axis=0)
        sm = jnp.sum(jnp.sum(hit * s_i, axis=0), axis=0)
        cnt_ref[pl.ds(rc, 1), :] = ct.reshape(1, 128)
        sums_ref[pl.ds(rc, 1), :] = sm.reshape(1, 128)
        return carry

    lax.fori_loop(0, K // 128, val_body, 0)

    cnt_row = cnt_ref[...].reshape(1, K)
    val_row = sums_ref[...].reshape(1, K) / jnp.maximum(cnt_row, 1.0)
    t_col = lax.broadcasted_iota(jnp.int32, (K, K), 0).astype(jnp.float32)
    r_row = lax.broadcasted_iota(jnp.int32, (K, K), 1).astype(jnp.float32)
    m2 = ((r_row <= t_col) & (t_col < r_row + cnt_row)).astype(jnp.float32)
    b = lax.dot_general(m2, f1w_ref[...], (((0,), (0,)), ((), ())),
                        preferred_element_type=jnp.float32)
    h = jnp.dot(val_row, b, preferred_element_type=jnp.float32)
    h = jnp.maximum(_ln(h + f1b_ref[...], g1_ref[...], be1_ref[...]), 0.0)
    h = jnp.dot(h, f2w_ref[...], preferred_element_type=jnp.float32)
    h = jnp.maximum(_ln(h + f2b_ref[...], g2_ref[...], be2_ref[...]), 0.0)
    h = jnp.dot(h, f3w_ref[...], preferred_element_type=jnp.float32)
    h = h + f3b_ref[...]
    o_ref[...] = 1.0 / (1.0 + jnp.exp(-h))


_head = pl.pallas_call(
    _head_kernel,
    out_shape=jax.ShapeDtypeStruct((1, 1), jnp.float32),
    scratch_shapes=[
        pltpu.VMEM((NC, 128), jnp.float32),
        pltpu.VMEM((NC, 128), jnp.float32),
        pltpu.VMEM((K // 128, 128), jnp.float32),
        pltpu.VMEM((K // 128, 128), jnp.float32),
    ],
)


def _gnn(xp, edges, zeros_rows, p, sfx):
    for l in ("1", "2", "3"):
        agg = _sc_scatter(xp, edges, zeros_rows)
        xp = _gin_layer(xp, agg, agg, p["w" + l + "a"], p["b" + l + "a"],
                        p["w" + l + "b"], p["b" + l + "b"])
    return xp


def kernel(x1, x2, edge_index1, edge_index2, w1a, b1a, w1b, b1b, w2a, b2a,
           w2b, b2b, w3a, b3a, w3b, b3b, wl, bl, fc1_w, fc1_b, fc2_w, fc2_b,
           fc3_w, fc3_b, g1, be1, g2, be2):
    f32 = jnp.float32
    xp1 = jnp.pad(x1, ((0, NPAD - N), (0, 0)))
    xp2 = jnp.pad(x2, ((0, NPAD - N), (0, 0)))
    zeros_rows = jnp.zeros((ROWS_PER_SUB, D), f32)
    p = {"w1a": w1a, "b1a": b1a.reshape(1, D), "w1b": w1b,
         "b1b": b1b.reshape(1, D), "w2a": w2a, "b2a": b2a.reshape(1, D),
         "w2b": w2b, "b2b": b2b.reshape(1, D), "w3a": w3a,
         "b3a": b3a.reshape(1, D), "w3b": w3b, "b3b": b3b.reshape(1, D)}

    h1 = _gnn(xp1, edge_index1.reshape(-1), zeros_rows, p, "g1")
    h2 = _gnn(xp2, edge_index2.reshape(-1), zeros_rows, p, "g2")

    out = _head(h1, h2, wl, bl.reshape(1, 2), fc1_w, fc1_b.reshape(1, D),
                fc2_w, fc2_b.reshape(1, D), fc3_w, fc3_b.reshape(1, 1),
                g1.reshape(1, D), be1.reshape(1, D), g2.reshape(1, D),
                be2.reshape(1, D))
    return out.reshape(1)
